# baseline (speedup 1.0000x reference)
import jax
import jax.numpy as jnp
from jax.experimental import pallas as pl
from jax.experimental.pallas import tpu as pltpu

_E = 320000


def _concat_body(gen_s, gen_t, gen_w, giv_s, giv_t,
                 out_s, out_t, out_w,
                 ones_v, s0, s1, s2, s3, s4, s5):
    c0 = pltpu.make_async_copy(gen_s, out_s.at[pl.ds(0, _E)], s0)
    c1 = pltpu.make_async_copy(gen_t, out_t.at[pl.ds(0, _E)], s1)
    c2 = pltpu.make_async_copy(gen_w, out_w.at[pl.ds(0, _E)], s2)
    c3 = pltpu.make_async_copy(giv_s, out_s.at[pl.ds(_E, _E)], s3)
    c4 = pltpu.make_async_copy(giv_t, out_t.at[pl.ds(_E, _E)], s4)
    c0.start()
    c1.start()
    c2.start()
    c3.start()
    c4.start()
    ones_v[...] = jnp.ones_like(ones_v)
    c5 = pltpu.make_async_copy(ones_v, out_w.at[pl.ds(_E, _E)], s5)
    c5.start()
    c0.wait()
    c1.wait()
    c2.wait()
    c3.wait()
    c4.wait()
    c5.wait()


@jax.jit
def _concat_dma(gen_s, gen_t, gen_w, giv_s, giv_t):
    hbm = pl.BlockSpec(memory_space=pltpu.MemorySpace.HBM)
    run = pl.pallas_call(
        _concat_body,
        out_shape=(
            jax.ShapeDtypeStruct((2 * _E,), jnp.int32),
            jax.ShapeDtypeStruct((2 * _E,), jnp.int32),
            jax.ShapeDtypeStruct((2 * _E,), jnp.float32),
        ),
        in_specs=[hbm] * 5,
        out_specs=(hbm, hbm, hbm),
        scratch_shapes=[pltpu.VMEM((_E,), jnp.float32)]
        + [pltpu.SemaphoreType.DMA] * 6,
    )
    return run(gen_s, gen_t, gen_w, giv_s, giv_t)


def kernel(gen_sources, gen_targets, gen_weights, given_sources,
           given_targets, node_embeddings):
    out_s, out_t, out_w = _concat_dma(
        gen_sources, gen_targets, gen_weights, given_sources, given_targets)
    return out_s, out_t, out_w, node_embeddings


# speedup vs baseline: 6.9694x; 6.9694x over previous
import jax
import jax.numpy as jnp
from jax.experimental import pallas as pl
from jax.experimental.pallas import tpu as pltpu

_E = 320000
_B = 32768
_G = -(-_E // _B)


def _concat_body(gen_s, gen_t, gen_w, giv_s, giv_t, out_s, out_t, out_w):
    out_s[0, :] = gen_s[...]
    out_s[1, :] = giv_s[...]
    out_t[0, :] = gen_t[...]
    out_t[1, :] = giv_t[...]
    out_w[0, :] = gen_w[...]
    out_w[1, :] = jnp.ones((_B,), jnp.float32)


@jax.jit
def _concat_dma(gen_s, gen_t, gen_w, giv_s, giv_t):
    in_spec = pl.BlockSpec((_B,), lambda i: (i,))
    out_spec = pl.BlockSpec((2, _B), lambda i: (0, i))
    out_s, out_t, out_w = pl.pallas_call(
        _concat_body,
        grid=(_G,),
        out_shape=(
            jax.ShapeDtypeStruct((2, _E), jnp.int32),
            jax.ShapeDtypeStruct((2, _E), jnp.int32),
            jax.ShapeDtypeStruct((2, _E), jnp.float32),
        ),
        in_specs=[in_spec] * 5,
        out_specs=(out_spec, out_spec, out_spec),
    )(gen_s, gen_t, gen_w, giv_s, giv_t)
    return (out_s.reshape(2 * _E), out_t.reshape(2 * _E),
            out_w.reshape(2 * _E))


def kernel(gen_sources, gen_targets, gen_weights, given_sources,
           given_targets, node_embeddings):
    out_s, out_t, out_w = _concat_dma(
        gen_sources, gen_targets, gen_weights, given_sources, given_targets)
    return out_s, out_t, out_w, node_embeddings


# speedup vs baseline: 7.0158x; 1.0067x over previous
import functools

import jax
import jax.numpy as jnp
from jax import lax
from jax.experimental import pallas as pl
from jax.experimental.pallas import tpu as pltpu
from jax.experimental.pallas import tpu_sc as plsc

_E = 320000
_NC = 2
_NS = 16
_NW = _NC * _NS
_C = _E // _NW
_L = 16

_mesh = plsc.VectorSubcoreMesh(core_axis_name="c", subcore_axis_name="s")


@functools.partial(
    pl.kernel,
    mesh=_mesh,
    out_type=(
        jax.ShapeDtypeStruct((2 * _E,), jnp.int32),
        jax.ShapeDtypeStruct((2 * _E,), jnp.int32),
        jax.ShapeDtypeStruct((2 * _E,), jnp.float32),
    ),
    scratch_types=[
        pltpu.VMEM((_C,), jnp.int32),
        pltpu.VMEM((_C,), jnp.int32),
        pltpu.VMEM((_C,), jnp.float32),
        pltpu.VMEM((_C,), jnp.int32),
        pltpu.VMEM((_C,), jnp.int32),
        pltpu.VMEM((_C,), jnp.float32),
        pltpu.SemaphoreType.DMA,
        pltpu.SemaphoreType.DMA,
        pltpu.SemaphoreType.DMA,
        pltpu.SemaphoreType.DMA,
        pltpu.SemaphoreType.DMA,
        pltpu.SemaphoreType.DMA,
        pltpu.SemaphoreType.DMA,
        pltpu.SemaphoreType.DMA,
        pltpu.SemaphoreType.DMA,
        pltpu.SemaphoreType.DMA,
        pltpu.SemaphoreType.DMA,
    ],
)
def _sc_concat(gen_s, gen_t, gen_w, giv_s, giv_t,
               out_s, out_t, out_w,
               b_gs, b_gt, b_gw, b_vs, b_vt, b_ones,
               s0, s1, s2, s3, s4, s5, s6, s7, s8, s9, s10):
    wid = lax.axis_index("s") * _NC + lax.axis_index("c")
    base = wid * _C
    gen = pl.ds(base, _C)
    giv = pl.ds(_E + base, _C)

    inbound = [
        pltpu.async_copy(gen_s.at[gen], b_gs, s0),
        pltpu.async_copy(gen_t.at[gen], b_gt, s1),
        pltpu.async_copy(gen_w.at[gen], b_gw, s2),
        pltpu.async_copy(giv_s.at[gen], b_vs, s3),
        pltpu.async_copy(giv_t.at[gen], b_vt, s4),
    ]

    ones16 = jnp.ones((_L,), jnp.float32)

    def _fill(i, carry):
        b_ones[pl.ds(i * _L, _L)] = ones16
        return carry

    lax.fori_loop(0, _C // _L, _fill, 0)

    ones_out = pltpu.async_copy(b_ones, out_w.at[giv], s5)

    inbound[0].wait()
    o0 = pltpu.async_copy(b_gs, out_s.at[gen], s6)
    inbound[1].wait()
    o1 = pltpu.async_copy(b_gt, out_t.at[gen], s7)
    inbound[2].wait()
    o2 = pltpu.async_copy(b_gw, out_w.at[gen], s8)
    inbound[3].wait()
    o3 = pltpu.async_copy(b_vs, out_s.at[giv], s9)
    inbound[4].wait()
    o4 = pltpu.async_copy(b_vt, out_t.at[giv], s10)

    ones_out.wait()
    o0.wait()
    o1.wait()
    o2.wait()
    o3.wait()
    o4.wait()


@jax.jit
def _run(gen_sources, gen_targets, gen_weights, given_sources,
         given_targets):
    return _sc_concat(gen_sources, gen_targets, gen_weights,
                      given_sources, given_targets)


def kernel(gen_sources, gen_targets, gen_weights, given_sources,
           given_targets, node_embeddings):
    out_s, out_t, out_w = _run(gen_sources, gen_targets, gen_weights,
                               given_sources, given_targets)
    return out_s, out_t, out_w, node_embeddings
